# Initial kernel scaffold; baseline (speedup 1.0000x reference)
import jax
import jax.numpy as jnp
from jax.experimental import pallas as pl


def kernel(x, edge_index, Wl1, bl1, Wr1, Wl2, bl2, Wr2, gW, gb):
    raise NotImplementedError("write your pallas kernel here")


# speedup vs baseline: 9.9044x; 9.9044x over previous
import functools

import jax
import jax.numpy as jnp
from jax import lax
from jax.experimental import pallas as pl
from jax.experimental.pallas import tpu as pltpu
from jax.experimental.pallas import tpu_sc as plsc

N = 10000
E = 320000
D = 128
NE = 8
K = 2
NC, NS = 2, 16
CW = 80
NG = 160
GR = E // (NG * CW)
NP = 10112
RT = NP // NS
B = 1000
NEG = -1e30


def _tile_chunks():
    offs, sizes = [], []
    o = 0
    while o < RT:
        s = min(CW, RT - o)
        offs.append(o)
        sizes.append(s)
        o += s
    return tuple(zip(offs, sizes))


def _sc_agg1_body(x_hbm, srcm, dstm, zrow, ones_hbm,
                  sums_out, cnts_out,
                  srcg, dstg, rows, dstb, acc, sem):
    core = lax.axis_index("c")
    sid = lax.axis_index("s")
    w = core * NS + sid
    r0 = sid * RT
    gpt = NG // (NC * NS)

    pltpu.sync_copy(zrow, rows)
    for o, s in _tile_chunks():
        pltpu.sync_copy(rows.at[pl.ds(0, s)], acc.at[pl.ds(r0 + o, s)])
    plsc.subcore_barrier()

    def group1(g2, carry):
        g = w * gpt + g2
        pltpu.sync_copy(srcm.at[g], srcg)
        pltpu.sync_copy(dstm.at[g], dstg)

        def batch(i, c2):
            for v in range(CW // 16):
                sl = pl.ds(v * 16, 16)
                dstb[sl] = dstg[i, sl]
            pltpu.async_copy(x_hbm.at[srcg.at[i]], rows, sem).wait()
            pltpu.sync_copy(rows, acc.at[dstb], add=True)
            return c2

        lax.fori_loop(0, GR, batch, 0)
        return carry

    lax.fori_loop(0, gpt, group1, 0)
    plsc.subcore_barrier()
    for o, s in _tile_chunks():
        pltpu.sync_copy(acc.at[pl.ds(r0 + o, s)], rows.at[pl.ds(0, s)])
        pltpu.sync_copy(rows.at[pl.ds(0, s)],
                        sums_out.at[core, pl.ds(r0 + o, s)])
    plsc.subcore_barrier()

    pltpu.sync_copy(zrow, rows)
    for o, s in _tile_chunks():
        pltpu.sync_copy(rows.at[pl.ds(0, s)], acc.at[pl.ds(r0 + o, s)])
    plsc.subcore_barrier()
    pltpu.sync_copy(ones_hbm, rows)

    def group2(g2, carry):
        g = w * gpt + g2
        pltpu.sync_copy(dstm.at[g], dstg)

        def batch(i, c2):
            for v in range(CW // 16):
                sl = pl.ds(v * 16, 16)
                dstb[sl] = dstg[i, sl]
            pltpu.sync_copy(rows, acc.at[dstb], add=True)
            return c2

        lax.fori_loop(0, GR, batch, 0)
        return carry

    lax.fori_loop(0, gpt, group2, 0)
    plsc.subcore_barrier()
    for o, s in _tile_chunks():
        pltpu.sync_copy(acc.at[pl.ds(r0 + o, s)], rows.at[pl.ds(0, s)])
        pltpu.sync_copy(rows.at[pl.ds(0, s)],
                        cnts_out.at[core, pl.ds(r0 + o, s)])


def _sc_agg1(x, srcm, dstm, zrow, ones):
    mesh = plsc.VectorSubcoreMesh(core_axis_name="c", subcore_axis_name="s")
    f = pl.kernel(
        _sc_agg1_body,
        out_type=[
            jax.ShapeDtypeStruct((NC, NP, D), jnp.float32),
            jax.ShapeDtypeStruct((NC, NP, D), jnp.float32),
        ],
        mesh=mesh,
        scratch_types=[
            pltpu.VMEM((GR, CW), jnp.int32),
            pltpu.VMEM((GR, CW), jnp.int32),
            pltpu.VMEM((CW, D), jnp.float32),
            pltpu.VMEM((CW,), jnp.int32),
            pltpu.VMEM_SHARED((NP, D), jnp.float32),
            pltpu.SemaphoreType.DMA,
        ],
    )
    return f(x, srcm, dstm, zrow, ones)


def _sc_agg2_body(h_hbm, srcm, dstm, ek2, zrow,
                  agg_out,
                  ekv, srcg, dstg, rows, dstb, acc, sem):
    core = lax.axis_index("c")
    sid = lax.axis_index("s")
    r0 = sid * RT
    pltpu.sync_copy(zrow, rows)
    for o, s in _tile_chunks():
        pltpu.sync_copy(rows.at[pl.ds(0, s)], acc.at[pl.ds(r0 + o, s)])
    pltpu.sync_copy(ek2.at[core], ekv)
    plsc.subcore_barrier()

    gpt = NG // NS
    GCW = GR * CW

    def group(g2, carry):
        g = sid * gpt + g2
        pltpu.sync_copy(srcm.at[g], srcg)
        pltpu.sync_copy(dstm.at[g], dstg)

        def idxbody(t, c2):
            sl = pl.ds(t * 16, 16)
            ev = plsc.load_gather(ekv, [dstg[sl]])
            srcg[sl] = srcg[sl] * NE + ev
            return c2

        lax.fori_loop(0, GCW // 16, idxbody, 0)

        def batch(i, c2):
            for v in range(CW // 16):
                dstb[pl.ds(v * 16, 16)] = dstg[pl.ds(i * CW + v * 16, 16)]
            pltpu.async_copy(h_hbm.at[srcg.at[pl.ds(i * CW, CW)]],
                             rows, sem).wait()
            pltpu.sync_copy(rows, acc.at[dstb], add=True)
            return c2

        lax.fori_loop(0, GR, batch, 0)
        return carry

    lax.fori_loop(0, gpt, group, 0)
    plsc.subcore_barrier()
    for o, s in _tile_chunks():
        pltpu.sync_copy(acc.at[pl.ds(r0 + o, s)], rows.at[pl.ds(0, s)])
        pltpu.sync_copy(rows.at[pl.ds(0, s)],
                        agg_out.at[core, pl.ds(r0 + o, s)])


def _sc_agg2(h2, srcm2, dstm2, ek2, zrow):
    mesh = plsc.VectorSubcoreMesh(core_axis_name="c", subcore_axis_name="s")
    f = pl.kernel(
        _sc_agg2_body,
        out_type=[jax.ShapeDtypeStruct((NC, NP, D), jnp.float32)],
        mesh=mesh,
        scratch_types=[
            pltpu.VMEM((N,), jnp.int32),
            pltpu.VMEM((GR * CW,), jnp.int32),
            pltpu.VMEM((GR * CW,), jnp.int32),
            pltpu.VMEM((CW, D), jnp.float32),
            pltpu.VMEM((CW,), jnp.int32),
            pltpu.VMEM_SHARED((NP, D), jnp.float32),
            pltpu.SemaphoreType.DMA,
        ],
        compiler_params=pltpu.CompilerParams(needs_layout_passes=False),
    )
    return f(h2, srcm2, dstm2, ek2, zrow)[0]


def _tc_h_body(p0, p1, c0, c1, xb, wl1, wr1, b1, gw, gbm,
               h_out, gi_out, gv_out, cnt_out):
    cnt = jnp.maximum(c0[0][:, :1] + c1[0][:, :1], 1.0)
    mean1 = (p0[0] + p1[0]) / cnt
    xv = xb[...]
    h = jnp.dot(mean1, wl1[...], preferred_element_type=jnp.float32)
    h = h + jnp.dot(xv, wr1[...], preferred_element_type=jnp.float32)
    h = h + b1[0:1, :]
    h_out[...] = jnp.maximum(h, 0.0)
    cnt_out[...] = jnp.broadcast_to(cnt, (B, NE))

    lg = jnp.dot(xv, gw[...], preferred_element_type=jnp.float32) + gbm[0:1, :]
    iota = lax.broadcasted_iota(jnp.int32, lg.shape, 1)
    m1 = jnp.max(lg, axis=1, keepdims=True)
    p = jnp.exp(lg - m1)
    s = jnp.sum(p, axis=1, keepdims=True)
    a1 = jnp.min(jnp.where(lg >= m1, iota, 128), axis=1, keepdims=True)
    lg2 = jnp.where(iota == a1, NEG, lg)
    m2 = jnp.max(lg2, axis=1, keepdims=True)
    a2 = jnp.min(jnp.where(lg2 >= m2, iota, 128), axis=1, keepdims=True)
    g1 = 1.0 / s
    g2 = jnp.exp(m2 - m1) / s
    i8 = lax.broadcasted_iota(jnp.int32, (B, NE), 1)
    gi_out[...] = jnp.where(i8 == 0, a1, jnp.where(i8 == 1, a2, 0))
    gv_out[...] = jnp.where(i8 == 0, g1, jnp.where(i8 == 1, g2, 0.0))


def _tc_h(sums, cnts, x, wl1c, wr1c, b1b, gwp, gbp):
    grid = (N // B,)
    return pl.pallas_call(
        _tc_h_body,
        grid=grid,
        in_specs=[
            pl.BlockSpec((1, B, D), lambda i: (0, i, 0)),
            pl.BlockSpec((1, B, D), lambda i: (1, i, 0)),
            pl.BlockSpec((1, B, D), lambda i: (0, i, 0)),
            pl.BlockSpec((1, B, D), lambda i: (1, i, 0)),
            pl.BlockSpec((B, D), lambda i: (i, 0)),
            pl.BlockSpec((D, NE * D), lambda i: (0, 0)),
            pl.BlockSpec((D, NE * D), lambda i: (0, 0)),
            pl.BlockSpec((8, NE * D), lambda i: (0, 0)),
            pl.BlockSpec((D, 128), lambda i: (0, 0)),
            pl.BlockSpec((8, 128), lambda i: (0, 0)),
        ],
        out_specs=[
            pl.BlockSpec((B, NE * D), lambda i: (i, 0)),
            pl.BlockSpec((B, NE), lambda i: (i, 0)),
            pl.BlockSpec((B, NE), lambda i: (i, 0)),
            pl.BlockSpec((B, NE), lambda i: (i, 0)),
        ],
        out_shape=[
            jax.ShapeDtypeStruct((N, NE * D), jnp.float32),
            jax.ShapeDtypeStruct((N, NE), jnp.int32),
            jax.ShapeDtypeStruct((N, NE), jnp.float32),
            jax.ShapeDtypeStruct((N, NE), jnp.float32),
        ],
    )(sums, sums, cnts, cnts, x, wl1c, wr1c, b1b, gwp, gbp)


def _tc_out_body(a0, a1, cb, hb, gi, gv, wl2, wr2, b2, out):
    cnt = cb[:, :1]
    h = hb[...]
    giv = gi[...]
    gvv = gv[...]
    acc = jnp.zeros((B, D), jnp.float32)
    for k in range(K):
        aggk = (a0, a1)[k][0]
        mk = aggk / cnt
        ekc = giv[:, k:k + 1]
        gvc = gvv[:, k:k + 1]
        g = jnp.zeros((B, D), jnp.float32)
        for e in range(NE):
            g = g + jnp.where(ekc == e, 1.0, 0.0) * h[:, e * D:(e + 1) * D]
        r = jnp.dot(mk, wl2[...], preferred_element_type=jnp.float32)
        r = r + jnp.dot(g, wr2[...], preferred_element_type=jnp.float32)
        r = r + b2[0:1, :]
        r = jnp.maximum(r, 0.0)
        for e in range(NE):
            acc = acc + jnp.where(ekc == e, gvc, 0.0) * r[:, e * D:(e + 1) * D]
    out[...] = acc


def _tc_out(agg, cnts, hh, gi, gvals, wl2c, wr2c, b2b):
    grid = (N // B,)
    return pl.pallas_call(
        _tc_out_body,
        grid=grid,
        in_specs=[
            pl.BlockSpec((1, B, D), lambda i: (0, i, 0)),
            pl.BlockSpec((1, B, D), lambda i: (1, i, 0)),
            pl.BlockSpec((B, NE), lambda i: (i, 0)),
            pl.BlockSpec((B, NE * D), lambda i: (i, 0)),
            pl.BlockSpec((B, NE), lambda i: (i, 0)),
            pl.BlockSpec((B, NE), lambda i: (i, 0)),
            pl.BlockSpec((D, NE * D), lambda i: (0, 0)),
            pl.BlockSpec((D, NE * D), lambda i: (0, 0)),
            pl.BlockSpec((8, NE * D), lambda i: (0, 0)),
        ],
        out_specs=pl.BlockSpec((B, D), lambda i: (i, 0)),
        out_shape=jax.ShapeDtypeStruct((N, D), jnp.float32),
    )(agg, agg, cnts, hh, gi, gvals, wl2c, wr2c, b2b)


def kernel(x, edge_index, Wl1, bl1, Wr1, Wl2, bl2, Wr2, gW, gb):
    src = edge_index[0]
    dst = edge_index[1]
    srcm = src.reshape(NG, GR, CW)
    dstm = dst.reshape(NG, GR, CW)

    zrow = jnp.zeros((CW, D), jnp.float32)
    ones = jnp.ones((CW, D), jnp.float32)

    sums, cnts = _sc_agg1(x, srcm, dstm, zrow, ones)

    wl1c = Wl1.transpose(2, 0, 1).reshape(D, NE * D)
    wr1c = Wr1.transpose(2, 0, 1).reshape(D, NE * D)
    b1b = jnp.broadcast_to(bl1.reshape(1, NE * D), (8, NE * D))
    gwp = jnp.concatenate([gW.T, jnp.zeros((D, 128 - NE), jnp.float32)], axis=1)
    gbp = jnp.broadcast_to(
        jnp.full((128,), NEG, jnp.float32).at[:NE].set(gb).reshape(1, 128),
        (8, 128))

    hh, gi, gvals, cntb = _tc_h(sums, cnts, x, wl1c, wr1c, b1b, gwp, gbp)

    ek2 = gi[:, :K].T.reshape(K, N)
    h2 = hh.reshape(N * NE, D)
    agg = _sc_agg2(h2, src.reshape(NG, GR * CW), dst.reshape(NG, GR * CW),
                   ek2, zrow)

    wl2c = Wl2.transpose(2, 0, 1).reshape(D, NE * D)
    wr2c = Wr2.transpose(2, 0, 1).reshape(D, NE * D)
    b2b = jnp.broadcast_to(bl2.reshape(1, NE * D), (8, NE * D))

    return _tc_out(agg, cntb, hh, gi, gvals, wl2c, wr2c, b2b)


# speedup vs baseline: 11.7459x; 1.1859x over previous
import functools

import jax
import jax.numpy as jnp
from jax import lax
from jax.experimental import pallas as pl
from jax.experimental.pallas import tpu as pltpu
from jax.experimental.pallas import tpu_sc as plsc

N = 10000
E = 320000
D = 128
NE = 8
K = 2
NC, NS = 2, 16
CW = 80
NG = 160
GR = E // (NG * CW)
NP = 10112
RT = NP // NS
B = 1000
NEG = -1e30


def _tile_chunks():
    offs, sizes = [], []
    o = 0
    while o < RT:
        s = min(CW, RT - o)
        offs.append(o)
        sizes.append(s)
        o += s
    return tuple(zip(offs, sizes))


def _sc_agg1_body(x_hbm, srcm, dstm, zrow, ones_hbm,
                  sums_out, cnts_out,
                  srcg, dstg, rows, rows2, dstb, dstb2, acc, sem, sem2):
    core = lax.axis_index("c")
    sid = lax.axis_index("s")
    w = core * NS + sid
    r0 = sid * RT
    gpt = NG // (NC * NS)

    pltpu.sync_copy(zrow, rows)
    for o, s in _tile_chunks():
        pltpu.sync_copy(rows.at[pl.ds(0, s)], acc.at[pl.ds(r0 + o, s)])
    plsc.subcore_barrier()

    def group1(g2, carry):
        g = w * gpt + g2
        pltpu.sync_copy(srcm.at[g], srcg)
        pltpu.sync_copy(dstm.at[g], dstg)

        def pair(i2, c2):
            i = i2 * 2
            h1 = pltpu.async_copy(x_hbm.at[srcg.at[i]], rows, sem)
            h2 = pltpu.async_copy(x_hbm.at[srcg.at[i + 1]], rows2, sem2)
            for v in range(CW // 16):
                sl = pl.ds(v * 16, 16)
                dstb[sl] = dstg[i, sl]
                dstb2[sl] = dstg[i + 1, sl]
            h1.wait()
            pltpu.sync_copy(rows, acc.at[dstb], add=True)
            h2.wait()
            pltpu.sync_copy(rows2, acc.at[dstb2], add=True)
            return c2

        lax.fori_loop(0, GR // 2, pair, 0)
        for v in range(CW // 16):
            sl = pl.ds(v * 16, 16)
            dstb[sl] = dstg[GR - 1, sl]
        pltpu.async_copy(x_hbm.at[srcg.at[GR - 1]], rows, sem).wait()
        pltpu.sync_copy(rows, acc.at[dstb], add=True)
        return carry

    lax.fori_loop(0, gpt, group1, 0)
    plsc.subcore_barrier()
    for o, s in _tile_chunks():
        pltpu.sync_copy(acc.at[pl.ds(r0 + o, s)], rows.at[pl.ds(0, s)])
        pltpu.sync_copy(rows.at[pl.ds(0, s)],
                        sums_out.at[core, pl.ds(r0 + o, s)])
    plsc.subcore_barrier()

    pltpu.sync_copy(zrow, rows)
    for o, s in _tile_chunks():
        pltpu.sync_copy(rows.at[pl.ds(0, s)], acc.at[pl.ds(r0 + o, s)])
    plsc.subcore_barrier()
    pltpu.sync_copy(ones_hbm, rows)

    def group2(g2, carry):
        g = w * gpt + g2
        pltpu.sync_copy(dstm.at[g], dstg)

        def batch(i, c2):
            for v in range(CW // 16):
                sl = pl.ds(v * 16, 16)
                dstb[sl] = dstg[i, sl]
            pltpu.sync_copy(rows, acc.at[dstb], add=True)
            return c2

        lax.fori_loop(0, GR, batch, 0)
        return carry

    lax.fori_loop(0, gpt, group2, 0)
    plsc.subcore_barrier()
    for o, s in _tile_chunks():
        pltpu.sync_copy(acc.at[pl.ds(r0 + o, s)], rows.at[pl.ds(0, s)])
        pltpu.sync_copy(rows.at[pl.ds(0, s)],
                        cnts_out.at[core, pl.ds(r0 + o, s)])


def _sc_agg1(x, srcm, dstm, zrow, ones):
    mesh = plsc.VectorSubcoreMesh(core_axis_name="c", subcore_axis_name="s")
    f = pl.kernel(
        _sc_agg1_body,
        out_type=[
            jax.ShapeDtypeStruct((NC, NP, D), jnp.float32),
            jax.ShapeDtypeStruct((NC, NP, D), jnp.float32),
        ],
        mesh=mesh,
        scratch_types=[
            pltpu.VMEM((GR, CW), jnp.int32),
            pltpu.VMEM((GR, CW), jnp.int32),
            pltpu.VMEM((CW, D), jnp.float32),
            pltpu.VMEM((CW, D), jnp.float32),
            pltpu.VMEM((CW,), jnp.int32),
            pltpu.VMEM((CW,), jnp.int32),
            pltpu.VMEM_SHARED((NP, D), jnp.float32),
            pltpu.SemaphoreType.DMA,
            pltpu.SemaphoreType.DMA,
        ],
    )
    return f(x, srcm, dstm, zrow, ones)


def _sc_agg2_body(h_hbm, srcm, dstm, ek2, zrow,
                  agg_out,
                  ekv, srcg, dstg, rows, rows2, dstb, dstb2, acc, sem, sem2):
    core = lax.axis_index("c")
    sid = lax.axis_index("s")
    r0 = sid * RT
    pltpu.sync_copy(zrow, rows)
    for o, s in _tile_chunks():
        pltpu.sync_copy(rows.at[pl.ds(0, s)], acc.at[pl.ds(r0 + o, s)])
    pltpu.sync_copy(ek2.at[core], ekv)
    plsc.subcore_barrier()

    gpt = NG // NS
    GCW = GR * CW

    def group(g2, carry):
        g = sid * gpt + g2
        pltpu.sync_copy(srcm.at[g], srcg)
        pltpu.sync_copy(dstm.at[g], dstg)

        def idxbody(t, c2):
            sl = pl.ds(t * 16, 16)
            ev = plsc.load_gather(ekv, [dstg[sl]])
            srcg[sl] = srcg[sl] * NE + ev
            return c2

        lax.fori_loop(0, GCW // 16, idxbody, 0)

        def pair(i2, c2):
            i = i2 * 2
            h1 = pltpu.async_copy(h_hbm.at[srcg.at[pl.ds(i * CW, CW)]],
                                  rows, sem)
            h2 = pltpu.async_copy(h_hbm.at[srcg.at[pl.ds((i + 1) * CW, CW)]],
                                  rows2, sem2)
            for v in range(CW // 16):
                dstb[pl.ds(v * 16, 16)] = dstg[pl.ds(i * CW + v * 16, 16)]
                dstb2[pl.ds(v * 16, 16)] = dstg[pl.ds((i + 1) * CW + v * 16, 16)]
            h1.wait()
            pltpu.sync_copy(rows, acc.at[dstb], add=True)
            h2.wait()
            pltpu.sync_copy(rows2, acc.at[dstb2], add=True)
            return c2

        lax.fori_loop(0, GR // 2, pair, 0)
        for v in range(CW // 16):
            dstb[pl.ds(v * 16, 16)] = dstg[pl.ds((GR - 1) * CW + v * 16, 16)]
        pltpu.async_copy(h_hbm.at[srcg.at[pl.ds((GR - 1) * CW, CW)]],
                         rows, sem).wait()
        pltpu.sync_copy(rows, acc.at[dstb], add=True)
        return carry

    lax.fori_loop(0, gpt, group, 0)
    plsc.subcore_barrier()
    for o, s in _tile_chunks():
        pltpu.sync_copy(acc.at[pl.ds(r0 + o, s)], rows.at[pl.ds(0, s)])
        pltpu.sync_copy(rows.at[pl.ds(0, s)],
                        agg_out.at[core, pl.ds(r0 + o, s)])


def _sc_agg2(h2, srcm2, dstm2, ek2, zrow):
    mesh = plsc.VectorSubcoreMesh(core_axis_name="c", subcore_axis_name="s")
    f = pl.kernel(
        _sc_agg2_body,
        out_type=[jax.ShapeDtypeStruct((NC, NP, D), jnp.float32)],
        mesh=mesh,
        scratch_types=[
            pltpu.VMEM((N,), jnp.int32),
            pltpu.VMEM((GR * CW,), jnp.int32),
            pltpu.VMEM((GR * CW,), jnp.int32),
            pltpu.VMEM((CW, D), jnp.float32),
            pltpu.VMEM((CW, D), jnp.float32),
            pltpu.VMEM((CW,), jnp.int32),
            pltpu.VMEM((CW,), jnp.int32),
            pltpu.VMEM_SHARED((NP, D), jnp.float32),
            pltpu.SemaphoreType.DMA,
            pltpu.SemaphoreType.DMA,
        ],
        compiler_params=pltpu.CompilerParams(needs_layout_passes=False),
    )
    return f(h2, srcm2, dstm2, ek2, zrow)[0]


def _tc_h_body(p0, p1, c0, c1, xb, wl1, wr1, b1, gw, gbm,
               h_out, gi_out, gv_out, cnt_out):
    cnt = jnp.maximum(c0[0][:, :1] + c1[0][:, :1], 1.0)
    mean1 = (p0[0] + p1[0]) / cnt
    xv = xb[...]
    h = jnp.dot(mean1, wl1[...], preferred_element_type=jnp.float32)
    h = h + jnp.dot(xv, wr1[...], preferred_element_type=jnp.float32)
    h = h + b1[0:1, :]
    h_out[...] = jnp.maximum(h, 0.0)
    cnt_out[...] = jnp.broadcast_to(cnt, (B, NE))

    lg = jnp.dot(xv, gw[...], preferred_element_type=jnp.float32) + gbm[0:1, :]
    iota = lax.broadcasted_iota(jnp.int32, lg.shape, 1)
    m1 = jnp.max(lg, axis=1, keepdims=True)
    p = jnp.exp(lg - m1)
    s = jnp.sum(p, axis=1, keepdims=True)
    a1 = jnp.min(jnp.where(lg >= m1, iota, 128), axis=1, keepdims=True)
    lg2 = jnp.where(iota == a1, NEG, lg)
    m2 = jnp.max(lg2, axis=1, keepdims=True)
    a2 = jnp.min(jnp.where(lg2 >= m2, iota, 128), axis=1, keepdims=True)
    g1 = 1.0 / s
    g2 = jnp.exp(m2 - m1) / s
    i8 = lax.broadcasted_iota(jnp.int32, (B, NE), 1)
    gi_out[...] = jnp.where(i8 == 0, a1, jnp.where(i8 == 1, a2, 0))
    gv_out[...] = jnp.where(i8 == 0, g1, jnp.where(i8 == 1, g2, 0.0))


def _tc_h(sums, cnts, x, wl1c, wr1c, b1b, gwp, gbp):
    grid = (N // B,)
    return pl.pallas_call(
        _tc_h_body,
        grid=grid,
        in_specs=[
            pl.BlockSpec((1, B, D), lambda i: (0, i, 0)),
            pl.BlockSpec((1, B, D), lambda i: (1, i, 0)),
            pl.BlockSpec((1, B, D), lambda i: (0, i, 0)),
            pl.BlockSpec((1, B, D), lambda i: (1, i, 0)),
            pl.BlockSpec((B, D), lambda i: (i, 0)),
            pl.BlockSpec((D, NE * D), lambda i: (0, 0)),
            pl.BlockSpec((D, NE * D), lambda i: (0, 0)),
            pl.BlockSpec((8, NE * D), lambda i: (0, 0)),
            pl.BlockSpec((D, 128), lambda i: (0, 0)),
            pl.BlockSpec((8, 128), lambda i: (0, 0)),
        ],
        out_specs=[
            pl.BlockSpec((B, NE * D), lambda i: (i, 0)),
            pl.BlockSpec((B, NE), lambda i: (i, 0)),
            pl.BlockSpec((B, NE), lambda i: (i, 0)),
            pl.BlockSpec((B, NE), lambda i: (i, 0)),
        ],
        out_shape=[
            jax.ShapeDtypeStruct((N, NE * D), jnp.float32),
            jax.ShapeDtypeStruct((N, NE), jnp.int32),
            jax.ShapeDtypeStruct((N, NE), jnp.float32),
            jax.ShapeDtypeStruct((N, NE), jnp.float32),
        ],
    )(sums, sums, cnts, cnts, x, wl1c, wr1c, b1b, gwp, gbp)


def _tc_out_body(a0, a1, cb, hb, gi, gv, wl2, wr2, b2, out):
    cnt = cb[:, :1]
    h = hb[...]
    giv = gi[...]
    gvv = gv[...]
    acc = jnp.zeros((B, D), jnp.float32)
    for k in range(K):
        aggk = (a0, a1)[k][0]
        mk = aggk / cnt
        ekc = giv[:, k:k + 1]
        gvc = gvv[:, k:k + 1]
        g = jnp.zeros((B, D), jnp.float32)
        for e in range(NE):
            g = g + jnp.where(ekc == e, 1.0, 0.0) * h[:, e * D:(e + 1) * D]
        r = jnp.dot(mk, wl2[...], preferred_element_type=jnp.float32)
        r = r + jnp.dot(g, wr2[...], preferred_element_type=jnp.float32)
        r = r + b2[0:1, :]
        r = jnp.maximum(r, 0.0)
        for e in range(NE):
            acc = acc + jnp.where(ekc == e, gvc, 0.0) * r[:, e * D:(e + 1) * D]
    out[...] = acc


def _tc_out(agg, cnts, hh, gi, gvals, wl2c, wr2c, b2b):
    grid = (N // B,)
    return pl.pallas_call(
        _tc_out_body,
        grid=grid,
        in_specs=[
            pl.BlockSpec((1, B, D), lambda i: (0, i, 0)),
            pl.BlockSpec((1, B, D), lambda i: (1, i, 0)),
            pl.BlockSpec((B, NE), lambda i: (i, 0)),
            pl.BlockSpec((B, NE * D), lambda i: (i, 0)),
            pl.BlockSpec((B, NE), lambda i: (i, 0)),
            pl.BlockSpec((B, NE), lambda i: (i, 0)),
            pl.BlockSpec((D, NE * D), lambda i: (0, 0)),
            pl.BlockSpec((D, NE * D), lambda i: (0, 0)),
            pl.BlockSpec((8, NE * D), lambda i: (0, 0)),
        ],
        out_specs=pl.BlockSpec((B, D), lambda i: (i, 0)),
        out_shape=jax.ShapeDtypeStruct((N, D), jnp.float32),
    )(agg, agg, cnts, hh, gi, gvals, wl2c, wr2c, b2b)


def kernel(x, edge_index, Wl1, bl1, Wr1, Wl2, bl2, Wr2, gW, gb):
    src = edge_index[0]
    dst = edge_index[1]
    srcm = src.reshape(NG, GR, CW)
    dstm = dst.reshape(NG, GR, CW)

    zrow = jnp.zeros((CW, D), jnp.float32)
    ones = jnp.ones((CW, D), jnp.float32)

    sums, cnts = _sc_agg1(x, srcm, dstm, zrow, ones)

    wl1c = Wl1.transpose(2, 0, 1).reshape(D, NE * D)
    wr1c = Wr1.transpose(2, 0, 1).reshape(D, NE * D)
    b1b = jnp.broadcast_to(bl1.reshape(1, NE * D), (8, NE * D))
    gwp = jnp.concatenate([gW.T, jnp.zeros((D, 128 - NE), jnp.float32)], axis=1)
    gbp = jnp.broadcast_to(
        jnp.full((128,), NEG, jnp.float32).at[:NE].set(gb).reshape(1, 128),
        (8, 128))

    hh, gi, gvals, cntb = _tc_h(sums, cnts, x, wl1c, wr1c, b1b, gwp, gbp)

    ek2 = gi[:, :K].T.reshape(K, N)
    h2 = hh.reshape(N * NE, D)
    agg = _sc_agg2(h2, src.reshape(NG, GR * CW), dst.reshape(NG, GR * CW),
                   ek2, zrow)

    wl2c = Wl2.transpose(2, 0, 1).reshape(D, NE * D)
    wr2c = Wr2.transpose(2, 0, 1).reshape(D, NE * D)
    b2b = jnp.broadcast_to(bl2.reshape(1, NE * D), (8, NE * D))

    return _tc_out(agg, cntb, hh, gi, gvals, wl2c, wr2c, b2b)


# speedup vs baseline: 14.0498x; 1.1961x over previous
import jax
import jax.numpy as jnp
from jax import lax
from jax.experimental import pallas as pl
from jax.experimental.pallas import tpu as pltpu
from jax.experimental.pallas import tpu_sc as plsc

N = 10000
E = 320000
D = 128
NE = 8
K = 2
NC, NS = 2, 16
CW = 80
NG = 160
GR = E // (NG * CW)
NP = 10112
RT = NP // NS
B = 1000
NEG = -1e30


def _tile_chunks():
    offs, sizes = [], []
    o = 0
    while o < RT:
        s = min(CW, RT - o)
        offs.append(o)
        sizes.append(s)
        o += s
    return tuple(zip(offs, sizes))


def _sc_agg1_body(x_hbm, srcm, dstm, zrow, ones_hbm,
                  sums_out, cnts_out,
                  srcg, dstg, rows, rows2, dstb, dstb2, acc, sem, sem2):
    core = lax.axis_index("c")
    sid = lax.axis_index("s")
    w = core * NS + sid
    r0 = sid * RT
    gpt = NG // (NC * NS)

    pltpu.sync_copy(zrow, rows)
    for o, s in _tile_chunks():
        pltpu.sync_copy(rows.at[pl.ds(0, s)], acc.at[pl.ds(r0 + o, s)])
    plsc.subcore_barrier()

    rowbufs = (rows, rows2)
    dstbufs = (dstb, dstb2)
    sems = (sem, sem2)

    def group1(g2, carry):
        g = w * gpt + g2
        pltpu.sync_copy(srcm.at[g], srcg)
        pltpu.sync_copy(dstm.at[g], dstg)

        handles = [
            pltpu.async_copy(x_hbm.at[srcg.at[i]], rowbufs[i], sems[i])
            for i in range(2)
        ]
        for i in range(GR):
            b = i % 2
            for v in range(CW // 16):
                sl = pl.ds(v * 16, 16)
                dstbufs[b][sl] = dstg[i, sl]
            handles[b].wait()
            pltpu.sync_copy(rowbufs[b], acc.at[dstbufs[b]], add=True)
            if i + 2 < GR:
                handles[b] = pltpu.async_copy(
                    x_hbm.at[srcg.at[i + 2]], rowbufs[b], sems[b])
        return carry

    lax.fori_loop(0, gpt, group1, 0)
    plsc.subcore_barrier()
    for o, s in _tile_chunks():
        pltpu.sync_copy(acc.at[pl.ds(r0 + o, s)], rows.at[pl.ds(0, s)])
        pltpu.sync_copy(rows.at[pl.ds(0, s)],
                        sums_out.at[core, pl.ds(r0 + o, s)])
    plsc.subcore_barrier()

    pltpu.sync_copy(zrow, rows)
    for o, s in _tile_chunks():
        pltpu.sync_copy(rows.at[pl.ds(0, s)], acc.at[pl.ds(r0 + o, s)])
    plsc.subcore_barrier()
    pltpu.sync_copy(ones_hbm, rows)

    def group2(g2, carry):
        g = w * gpt + g2
        pltpu.sync_copy(dstm.at[g], dstg)

        def batch(i, c2):
            for v in range(CW // 16):
                sl = pl.ds(v * 16, 16)
                dstb[sl] = dstg[i, sl]
            pltpu.sync_copy(rows, acc.at[dstb], add=True)
            return c2

        lax.fori_loop(0, GR, batch, 0)
        return carry

    lax.fori_loop(0, gpt, group2, 0)
    plsc.subcore_barrier()
    for o, s in _tile_chunks():
        pltpu.sync_copy(acc.at[pl.ds(r0 + o, s)], rows.at[pl.ds(0, s)])
        pltpu.sync_copy(rows.at[pl.ds(0, s)],
                        cnts_out.at[core, pl.ds(r0 + o, s)])


def _sc_agg1(x, srcm, dstm, zrow, ones):
    mesh = plsc.VectorSubcoreMesh(core_axis_name="c", subcore_axis_name="s")
    f = pl.kernel(
        _sc_agg1_body,
        out_type=[
            jax.ShapeDtypeStruct((NC, NP, D), jnp.float32),
            jax.ShapeDtypeStruct((NC, NP, D), jnp.float32),
        ],
        mesh=mesh,
        scratch_types=[
            pltpu.VMEM((GR, CW), jnp.int32),
            pltpu.VMEM((GR, CW), jnp.int32),
            pltpu.VMEM((CW, D), jnp.float32),
            pltpu.VMEM((CW, D), jnp.float32),
            pltpu.VMEM((CW,), jnp.int32),
            pltpu.VMEM((CW,), jnp.int32),
            pltpu.VMEM_SHARED((NP, D), jnp.float32),
            pltpu.SemaphoreType.DMA,
            pltpu.SemaphoreType.DMA,
        ],
    )
    return f(x, srcm, dstm, zrow, ones)


def _sc_agg2_body(h_hbm, srcm, dstm, ek2, zrow,
                  agg_out,
                  ekv, srcg, dstg, rows, rows2, dstb, dstb2, acc, sem, sem2):
    core = lax.axis_index("c")
    sid = lax.axis_index("s")
    r0 = sid * RT
    pltpu.sync_copy(zrow, rows)
    for o, s in _tile_chunks():
        pltpu.sync_copy(rows.at[pl.ds(0, s)], acc.at[pl.ds(r0 + o, s)])
    pltpu.sync_copy(ek2.at[core], ekv)
    plsc.subcore_barrier()

    gpt = NG // NS
    GCW = GR * CW

    def group(g2, carry):
        g = sid * gpt + g2
        pltpu.sync_copy(srcm.at[g], srcg)
        pltpu.sync_copy(dstm.at[g], dstg)

        def idxbody(t, c2):
            sl = pl.ds(t * 16, 16)
            ev = plsc.load_gather(ekv, [dstg[sl]])
            srcg[sl] = srcg[sl] * NE + ev
            return c2

        lax.fori_loop(0, GCW // 16, idxbody, 0)

        rowbufs = (rows, rows2)
        dstbufs = (dstb, dstb2)
        sems = (sem, sem2)
        handles = [
            pltpu.async_copy(h_hbm.at[srcg.at[pl.ds(i * CW, CW)]],
                             rowbufs[i], sems[i])
            for i in range(2)
        ]
        for i in range(GR):
            b = i % 2
            for v in range(CW // 16):
                dstbufs[b][pl.ds(v * 16, 16)] = dstg[pl.ds(i * CW + v * 16, 16)]
            handles[b].wait()
            pltpu.sync_copy(rowbufs[b], acc.at[dstbufs[b]], add=True)
            if i + 2 < GR:
                handles[b] = pltpu.async_copy(
                    h_hbm.at[srcg.at[pl.ds((i + 2) * CW, CW)]],
                    rowbufs[b], sems[b])
        return carry

    lax.fori_loop(0, gpt, group, 0)
    plsc.subcore_barrier()
    for o, s in _tile_chunks():
        pltpu.sync_copy(acc.at[pl.ds(r0 + o, s)], rows.at[pl.ds(0, s)])
        pltpu.sync_copy(rows.at[pl.ds(0, s)],
                        agg_out.at[core, pl.ds(r0 + o, s)])


def _sc_agg2(h2, srcm2, dstm2, ek2, zrow):
    mesh = plsc.VectorSubcoreMesh(core_axis_name="c", subcore_axis_name="s")
    f = pl.kernel(
        _sc_agg2_body,
        out_type=[jax.ShapeDtypeStruct((NC, NP, D), jnp.float32)],
        mesh=mesh,
        scratch_types=[
            pltpu.VMEM((N,), jnp.int32),
            pltpu.VMEM((GR * CW,), jnp.int32),
            pltpu.VMEM((GR * CW,), jnp.int32),
            pltpu.VMEM((CW, D), jnp.float32),
            pltpu.VMEM((CW, D), jnp.float32),
            pltpu.VMEM((CW,), jnp.int32),
            pltpu.VMEM((CW,), jnp.int32),
            pltpu.VMEM_SHARED((NP, D), jnp.float32),
            pltpu.SemaphoreType.DMA,
            pltpu.SemaphoreType.DMA,
        ],
        compiler_params=pltpu.CompilerParams(needs_layout_passes=False),
    )
    return f(h2, srcm2, dstm2, ek2, zrow)[0]


def _tc_h_body(p0, p1, c0, c1, xb, wl1, wr1, b1, gw, gbm,
               h_out, gi_out, gv_out, cnt_out):
    cnt = jnp.maximum(c0[0][:, :1] + c1[0][:, :1], 1.0)
    mean1 = (p0[0] + p1[0]) / cnt
    xv = xb[...]
    h = jnp.dot(mean1, wl1[...], preferred_element_type=jnp.float32)
    h = h + jnp.dot(xv, wr1[...], preferred_element_type=jnp.float32)
    h = h + b1[0:1, :]
    h_out[...] = jnp.maximum(h, 0.0)
    cnt_out[...] = jnp.broadcast_to(cnt, (B, NE))

    lg = jnp.dot(xv, gw[...], preferred_element_type=jnp.float32) + gbm[0:1, :]
    iota = lax.broadcasted_iota(jnp.int32, lg.shape, 1)
    m1 = jnp.max(lg, axis=1, keepdims=True)
    p = jnp.exp(lg - m1)
    s = jnp.sum(p, axis=1, keepdims=True)
    a1 = jnp.min(jnp.where(lg >= m1, iota, 128), axis=1, keepdims=True)
    lg2 = jnp.where(iota == a1, NEG, lg)
    m2 = jnp.max(lg2, axis=1, keepdims=True)
    a2 = jnp.min(jnp.where(lg2 >= m2, iota, 128), axis=1, keepdims=True)
    g1 = 1.0 / s
    g2 = jnp.exp(m2 - m1) / s
    i8 = lax.broadcasted_iota(jnp.int32, (B, NE), 1)
    gi_out[...] = jnp.where(i8 == 0, a1, jnp.where(i8 == 1, a2, 0))
    gv_out[...] = jnp.where(i8 == 0, g1, jnp.where(i8 == 1, g2, 0.0))


def _tc_h(sums, cnts, x, wl1c, wr1c, b1b, gwp, gbp):
    grid = (N // B,)
    return pl.pallas_call(
        _tc_h_body,
        grid=grid,
        in_specs=[
            pl.BlockSpec((1, B, D), lambda i: (0, i, 0)),
            pl.BlockSpec((1, B, D), lambda i: (1, i, 0)),
            pl.BlockSpec((1, B, D), lambda i: (0, i, 0)),
            pl.BlockSpec((1, B, D), lambda i: (1, i, 0)),
            pl.BlockSpec((B, D), lambda i: (i, 0)),
            pl.BlockSpec((D, NE * D), lambda i: (0, 0)),
            pl.BlockSpec((D, NE * D), lambda i: (0, 0)),
            pl.BlockSpec((8, NE * D), lambda i: (0, 0)),
            pl.BlockSpec((D, 128), lambda i: (0, 0)),
            pl.BlockSpec((8, 128), lambda i: (0, 0)),
        ],
        out_specs=[
            pl.BlockSpec((B, NE * D), lambda i: (i, 0)),
            pl.BlockSpec((B, NE), lambda i: (i, 0)),
            pl.BlockSpec((B, NE), lambda i: (i, 0)),
            pl.BlockSpec((B, NE), lambda i: (i, 0)),
        ],
        out_shape=[
            jax.ShapeDtypeStruct((N, NE * D), jnp.float32),
            jax.ShapeDtypeStruct((N, NE), jnp.int32),
            jax.ShapeDtypeStruct((N, NE), jnp.float32),
            jax.ShapeDtypeStruct((N, NE), jnp.float32),
        ],
    )(sums, sums, cnts, cnts, x, wl1c, wr1c, b1b, gwp, gbp)


def _tc_out_body(a0, a1, cb, hb, gi, gv, wl2, wr2, b2, out):
    cnt = cb[:, :1]
    h = hb[...]
    giv = gi[...]
    gvv = gv[...]
    acc = jnp.zeros((B, D), jnp.float32)
    for k in range(K):
        aggk = (a0, a1)[k][0]
        mk = aggk / cnt
        ekc = giv[:, k:k + 1]
        gvc = gvv[:, k:k + 1]
        g = jnp.zeros((B, D), jnp.float32)
        for e in range(NE):
            g = g + jnp.where(ekc == e, 1.0, 0.0) * h[:, e * D:(e + 1) * D]
        r = jnp.dot(mk, wl2[...], preferred_element_type=jnp.float32)
        r = r + jnp.dot(g, wr2[...], preferred_element_type=jnp.float32)
        r = r + b2[0:1, :]
        r = jnp.maximum(r, 0.0)
        for e in range(NE):
            acc = acc + jnp.where(ekc == e, gvc, 0.0) * r[:, e * D:(e + 1) * D]
    out[...] = acc


def _tc_out(agg, cnts, hh, gi, gvals, wl2c, wr2c, b2b):
    grid = (N // B,)
    return pl.pallas_call(
        _tc_out_body,
        grid=grid,
        in_specs=[
            pl.BlockSpec((1, B, D), lambda i: (0, i, 0)),
            pl.BlockSpec((1, B, D), lambda i: (1, i, 0)),
            pl.BlockSpec((B, NE), lambda i: (i, 0)),
            pl.BlockSpec((B, NE * D), lambda i: (i, 0)),
            pl.BlockSpec((B, NE), lambda i: (i, 0)),
            pl.BlockSpec((B, NE), lambda i: (i, 0)),
            pl.BlockSpec((D, NE * D), lambda i: (0, 0)),
            pl.BlockSpec((D, NE * D), lambda i: (0, 0)),
            pl.BlockSpec((8, NE * D), lambda i: (0, 0)),
        ],
        out_specs=pl.BlockSpec((B, D), lambda i: (i, 0)),
        out_shape=jax.ShapeDtypeStruct((N, D), jnp.float32),
    )(agg, agg, cnts, hh, gi, gvals, wl2c, wr2c, b2b)


def kernel(x, edge_index, Wl1, bl1, Wr1, Wl2, bl2, Wr2, gW, gb):
    src = edge_index[0]
    dst = edge_index[1]
    srcm = src.reshape(NG, GR, CW)
    dstm = dst.reshape(NG, GR, CW)

    zrow = jnp.zeros((CW, D), jnp.float32)
    ones = jnp.ones((CW, D), jnp.float32)

    sums, cnts = _sc_agg1(x, srcm, dstm, zrow, ones)

    wl1c = Wl1.transpose(2, 0, 1).reshape(D, NE * D)
    wr1c = Wr1.transpose(2, 0, 1).reshape(D, NE * D)
    b1b = jnp.broadcast_to(bl1.reshape(1, NE * D), (8, NE * D))
    gwp = jnp.concatenate([gW.T, jnp.zeros((D, 128 - NE), jnp.float32)], axis=1)
    gbp = jnp.broadcast_to(
        jnp.full((128,), NEG, jnp.float32).at[:NE].set(gb).reshape(1, 128),
        (8, 128))

    hh, gi, gvals, cntb = _tc_h(sums, cnts, x, wl1c, wr1c, b1b, gwp, gbp)

    ek2 = gi[:, :K].T.reshape(K, N)
    h2 = hh.reshape(N * NE, D)
    agg = _sc_agg2(h2, src.reshape(NG, GR * CW), dst.reshape(NG, GR * CW),
                   ek2, zrow)

    wl2c = Wl2.transpose(2, 0, 1).reshape(D, NE * D)
    wr2c = Wr2.transpose(2, 0, 1).reshape(D, NE * D)
    b2b = jnp.broadcast_to(bl2.reshape(1, NE * D), (8, NE * D))

    return _tc_out(agg, cntb, hh, gi, gvals, wl2c, wr2c, b2b)
